# baseline (speedup 1.0000x reference)
import functools

import jax
import jax.numpy as jnp
from jax import lax
from jax.experimental import pallas as pl
from jax.experimental.pallas import tpu as pltpu
from jax.experimental.pallas import tpu_sc as plsc

_H = 128
_N = 10000
_E = 640000
_NCORES = 2
_NSUB = 16
_NW = _NCORES * _NSUB
_K = 80
_CHUNKS = _E // (_NW * _K)
_GC = 50
_NGROUP = _CHUNKS // _GC
_NZ = 10
_RPZ = _N // _NZ

_KC = 128
_CPT = 157
_EPAD = _NW * _CPT * _KC - _E
_NP = _N + 16


def _segsum_body(table_hbm, src_hbm, dst_hbm, zsum_hbm,
                 sums_hbm,
                 sidx, didx, rows_a, rows_b, rows_c, acc_sh,
                 gsem_a, gsem_b, gsem_c, ssem_a, ssem_b, ssem_c):
    c = lax.axis_index("c")
    s = lax.axis_index("s")
    wid = s * _NCORES + c

    @pl.when(s < _NZ)
    def _():
        row0 = s * _RPZ
        pltpu.sync_copy(zsum_hbm.at[pl.ds(row0, _RPZ), :],
                        acc_sh.at[pl.ds(row0, _RPZ), :])

    plsc.subcore_barrier()

    bufs = ((rows_a, gsem_a, ssem_a), (rows_b, gsem_b, ssem_b),
            (rows_c, gsem_c, ssem_c))

    def _group(g, _):
        pltpu.sync_copy(src_hbm.at[wid, g], sidx)
        pltpu.sync_copy(dst_hbm.at[wid, g], didx)
        pltpu.async_copy(table_hbm.at[sidx.at[0]], rows_a, gsem_a)
        pltpu.async_copy(table_hbm.at[sidx.at[1]], rows_b, gsem_b)
        for j in range(_GC):
            buf, gsem, ssem = bufs[j % 3]
            if j + 2 < _GC:
                nbuf, ngsem, nssem = bufs[(j + 2) % 3]
                if j >= 1:
                    pltpu.make_async_copy(
                        nbuf, acc_sh.at[didx.at[j - 1]], nssem).wait()
                pltpu.async_copy(table_hbm.at[sidx.at[j + 2]], nbuf, ngsem)
            pltpu.make_async_copy(table_hbm.at[sidx.at[j]], buf, gsem).wait()
            pltpu.async_copy(buf, acc_sh.at[didx.at[j]], ssem, add=True)
        for j in range(_GC - 3, _GC):
            buf, _, ssem = bufs[j % 3]
            pltpu.make_async_copy(buf, acc_sh.at[didx.at[j]], ssem).wait()
        return 0

    lax.fori_loop(0, _NGROUP, _group, 0)
    plsc.subcore_barrier()

    @pl.when(s < _NZ)
    def _():
        row0 = s * _RPZ
        pltpu.sync_copy(acc_sh.at[pl.ds(row0, _RPZ), :],
                        sums_hbm.at[c, pl.ds(row0, _RPZ), :])


@jax.jit
def _segsum(table, src3, dst3, zsum):
    mesh = plsc.VectorSubcoreMesh(core_axis_name="c", subcore_axis_name="s")
    return pl.kernel(
        _segsum_body,
        out_type=jax.ShapeDtypeStruct((_NCORES, _N, _H), jnp.float32),
        mesh=mesh,
        scratch_types=[
            pltpu.VMEM((_GC, _K), jnp.int32),
            pltpu.VMEM((_GC, _K), jnp.int32),
            pltpu.VMEM((_K, _H), jnp.float32),
            pltpu.VMEM((_K, _H), jnp.float32),
            pltpu.VMEM((_K, _H), jnp.float32),
            pltpu.VMEM_SHARED((_N, _H), jnp.float32),
        ] + [pltpu.SemaphoreType.DMA] * 6,
    )(table, src3, dst3, zsum)


def _counts_body(dstv_hbm, dstc_hbm, zcnt_hbm,
                 cntv_hbm, cntc_hbm,
                 didx, ones, cnt_sh, stg_sem, csem):
    c = lax.axis_index("c")
    s = lax.axis_index("s")
    wid = s * _NCORES + c

    for i in range(_KC // 16):
        ones[pl.ds(i * 16, 16)] = jnp.ones((16,), jnp.float32)

    for d_hbm, out_hbm in ((dstv_hbm, cntv_hbm), (dstc_hbm, cntc_hbm)):
        @pl.when(s == 0)
        def _():
            pltpu.sync_copy(zcnt_hbm, cnt_sh)

        pltpu.async_copy(d_hbm.at[wid], didx, stg_sem)
        pltpu.make_async_copy(d_hbm.at[wid], didx, stg_sem).wait()
        plsc.subcore_barrier()

        def _fire(j, _):
            pltpu.async_copy(ones, cnt_sh.at[didx.at[j]], csem, add=True)
            return 0

        def _fire_drain(j, _):
            pltpu.async_copy(ones, cnt_sh.at[didx.at[j]], csem, add=True)
            pltpu.make_async_copy(ones, cnt_sh.at[didx.at[j - 8]], csem).wait()
            return 0

        lax.fori_loop(0, 8, _fire, 0)
        lax.fori_loop(8, _CPT, _fire_drain, 0)
        lax.fori_loop(_CPT - 8, _CPT, lambda j, _: (
            pltpu.make_async_copy(ones, cnt_sh.at[didx.at[j]], csem).wait(),
            0)[1], 0)
        plsc.subcore_barrier()

        @pl.when(s == 0)
        def _():
            pltpu.sync_copy(cnt_sh, out_hbm.at[c])

        plsc.subcore_barrier()


@jax.jit
def _counts(dstv4, dstc4, zcnt):
    mesh = plsc.VectorSubcoreMesh(core_axis_name="c", subcore_axis_name="s")
    return pl.kernel(
        _counts_body,
        out_type=(jax.ShapeDtypeStruct((_NCORES, _NP), jnp.float32),
                  jax.ShapeDtypeStruct((_NCORES, _NP), jnp.float32)),
        mesh=mesh,
        scratch_types=[
            pltpu.VMEM((_CPT, _KC), jnp.int32),
            pltpu.VMEM((_KC,), jnp.float32),
            pltpu.VMEM_SHARED((_NP,), jnp.float32),
            pltpu.SemaphoreType.DMA,
            pltpu.SemaphoreType.DMA,
        ],
    )(dstv4, dstc4, zcnt)


_BLK = 2000
_GRID = _N // _BLK


def _ln(x, g, b):
    m = jnp.mean(x, axis=-1, keepdims=True)
    v = jnp.mean((x - m) ** 2, axis=-1, keepdims=True)
    return (x - m) * jax.lax.rsqrt(v + 1e-5) * g + b


def _proj_body(xvar, xcon, wvT, bv, gv, bvn, wcT, bc, gc, bcn, xv_o, xc_o):
    hv = jax.nn.relu(jnp.dot(xvar[...], wvT[...],
                             preferred_element_type=jnp.float32) + bv[...])
    xv_o[...] = _ln(hv, gv[...], bvn[...])
    hc = jax.nn.relu(jnp.dot(xcon[...], wcT[...],
                             preferred_element_type=jnp.float32) + bc[...])
    xc_o[...] = _ln(hc, gc[...], bcn[...])


def _v2c_body(sums, cnts, xcon, xc, wmT, bm, wuT, wub, bu, gl, bl,
              wg1T, wg2T, bg, out):
    sp = sums[...]
    S = sp[0] + sp[1]
    cp = cnts[...]
    cnt = cp[0] + cp[1]
    agg = (jnp.dot(S, wmT[...], preferred_element_type=jnp.float32)
           + cnt * bm[...]) / (cnt + 1e-6)
    h = (jnp.dot(agg, wuT[...], preferred_element_type=jnp.float32)
         + xcon[...] * wub[...] + bu[...])
    con_new = _ln(jax.nn.relu(h), gl[...], bl[...])
    xc_v = xc[...]
    g = jax.nn.sigmoid(
        jnp.dot(xc_v, wg1T[...], preferred_element_type=jnp.float32)
        + jnp.dot(con_new, wg2T[...], preferred_element_type=jnp.float32)
        + bg[...])
    out[...] = g * con_new + (1.0 - g) * xc_v


def _c2v_body(sums, cnts, xv, wmT, bm, gl, bl, wg1T, wg2T, bg, out):
    sp = sums[...]
    S = sp[0] + sp[1]
    cp = cnts[...]
    cnt = cp[0] + cp[1]
    agg = (jnp.dot(S, wmT[...], preferred_element_type=jnp.float32)
           + cnt * bm[...]) / (cnt + 1e-6)
    var_new = _ln(jax.nn.relu(agg), gl[...], bl[...])
    xv_v = xv[...]
    g = jax.nn.sigmoid(
        jnp.dot(xv_v, wg1T[...], preferred_element_type=jnp.float32)
        + jnp.dot(var_new, wg2T[...], preferred_element_type=jnp.float32)
        + bg[...])
    out[...] = g * var_new + (1.0 - g) * xv_v


def _rows(blk_last):
    return pl.BlockSpec((_BLK, blk_last), lambda i: (i, 0))


def _prows(blk_last):
    return pl.BlockSpec((2, _BLK, blk_last), lambda i: (0, i, 0))


def _w(shape):
    return pl.BlockSpec(shape, lambda i: tuple(0 for _ in shape))


@jax.jit
def _tc_proj(x_var, x_con, wvT, bv, gv, bvn, wcT, bc, gc, bcn):
    return pl.pallas_call(
        _proj_body,
        grid=(_GRID,),
        in_specs=[_rows(4), _rows(1),
                  _w((4, _H)), _w((1, _H)), _w((1, _H)), _w((1, _H)),
                  _w((1, _H)), _w((1, _H)), _w((1, _H)), _w((1, _H))],
        out_specs=[_rows(_H), _rows(_H)],
        out_shape=[jax.ShapeDtypeStruct((_N, _H), jnp.float32),
                   jax.ShapeDtypeStruct((_N, _H), jnp.float32)],
    )(x_var, x_con, wvT, bv, gv, bvn, wcT, bc, gc, bcn)


@jax.jit
def _tc_v2c(sums, cnts, x_con, xc, wmT, bm, wuT, wub, bu, gl, bl,
            wg1T, wg2T, bg):
    return pl.pallas_call(
        _v2c_body,
        grid=(_GRID,),
        in_specs=[_prows(_H), _prows(1), _rows(1), _rows(_H),
                  _w((_H, _H)), _w((1, _H)), _w((_H, _H)), _w((1, _H)),
                  _w((1, _H)), _w((1, _H)), _w((1, _H)),
                  _w((_H, _H)), _w((_H, _H)), _w((1, _H))],
        out_specs=_rows(_H),
        out_shape=jax.ShapeDtypeStruct((_N, _H), jnp.float32),
    )(sums, cnts, x_con, xc, wmT, bm, wuT, wub, bu, gl, bl, wg1T, wg2T, bg)


@jax.jit
def _tc_c2v(sums, cnts, xv, wmT, bm, gl, bl, wg1T, wg2T, bg):
    return pl.pallas_call(
        _c2v_body,
        grid=(_GRID,),
        in_specs=[_prows(_H), _prows(1), _rows(_H),
                  _w((_H, _H)), _w((1, _H)), _w((1, _H)), _w((1, _H)),
                  _w((_H, _H)), _w((_H, _H)), _w((1, _H))],
        out_specs=_rows(_H),
        out_shape=jax.ShapeDtypeStruct((_N, _H), jnp.float32),
    )(sums, cnts, xv, wmT, bm, gl, bl, wg1T, wg2T, bg)


def kernel(x_var, x_con, e_var_con, e_con_var, params):
    p = params
    r2 = lambda b: b.reshape(1, _H)

    wvT = p["var_proj"][0].T
    bv = r2(p["var_proj"][1])
    gv, bvn = r2(p["var_norm"][0]), r2(p["var_norm"][1])
    wcT = p["con_proj"][0].T
    bc = r2(p["con_proj"][1])
    gc, bcn = r2(p["con_norm"][0]), r2(p["con_norm"][1])

    wmT = p["v2c_msg"][0].T
    bm = r2(p["v2c_msg"][1])
    wu = p["v2c_upd"][0]
    wuT = wu[:, :_H].T
    wub = wu[:, _H].reshape(1, _H)
    bu = r2(p["v2c_upd"][1])
    gl_c, bl_c = r2(p["v2c_ln"][0]), r2(p["v2c_ln"][1])
    wg_c = p["con_gate"][0]
    wg1T_c, wg2T_c = wg_c[:, :_H].T, wg_c[:, _H:].T
    bg_c = r2(p["con_gate"][1])

    wm2T = p["c2v_msg"][0].T
    bm2 = r2(p["c2v_msg"][1])
    gl_v, bl_v = r2(p["c2v_ln"][0]), r2(p["c2v_ln"][1])
    wg_v = p["var_gate"][0]
    wg1T_v, wg2T_v = wg_v[:, :_H].T, wg_v[:, _H:].T
    bg_v = r2(p["var_gate"][1])

    eshape = (_NW, _NGROUP, _GC, _K)
    src_v = e_var_con[0].astype(jnp.int32).reshape(eshape)
    dst_v = e_var_con[1].astype(jnp.int32).reshape(eshape)
    src_c = e_con_var[0].astype(jnp.int32).reshape(eshape)
    dst_c = e_con_var[1].astype(jnp.int32).reshape(eshape)
    zsum = jnp.zeros((_N, _H), jnp.float32)
    zcnt = jnp.zeros((_NP,), jnp.float32)

    pad = jnp.full((_EPAD,), _N, jnp.int32)
    cshape = (_NW, _CPT, _KC)
    dv4 = jnp.concatenate([e_var_con[1].astype(jnp.int32), pad]).reshape(cshape)
    dc4 = jnp.concatenate([e_con_var[1].astype(jnp.int32), pad]).reshape(cshape)
    cv, cc = _counts(dv4, dc4, zcnt)
    cnts_v = cv[:, :_N, None]
    cnts_c = cc[:, :_N, None]

    xv, xc = _tc_proj(x_var, x_con, wvT, bv, gv, bvn, wcT, bc, gc, bcn)

    for _ in range(2):
        sums = _segsum(xv, src_v, dst_v, zsum)
        xc = _tc_v2c(sums, cnts_v, x_con, xc,
                     wmT, bm, wuT, wub, bu, gl_c, bl_c,
                     wg1T_c, wg2T_c, bg_c)
        sums2 = _segsum(xc, src_c, dst_c, zsum)
        xv = _tc_c2v(sums2, cnts_c, xv,
                     wm2T, bm2, gl_v, bl_v, wg1T_v, wg2T_v, bg_v)
    return xv


# speedup vs baseline: 1.0008x; 1.0008x over previous
import jax
import jax.numpy as jnp
from jax import lax
from jax.experimental import pallas as pl
from jax.experimental.pallas import tpu as pltpu
from jax.experimental.pallas import tpu_sc as plsc

_H = 128
_N = 10000
_E = 640000
_NCORES = 2
_NSUB = 16
_NW = _NCORES * _NSUB
_K = 80
_CHUNKS = _E // (_NW * _K)
_GC = 50
_NGROUP = _CHUNKS // _GC
_NZ = 10
_RPZ = _N // _NZ

_KC = 128
_CPT = 157
_EPAD = _NW * _CPT * _KC - _E
_NP = _N + 16


def _segsum_body(table_hbm, src_hbm, dst_hbm, zsum_hbm,
                 sums_hbm,
                 sidx, didx, rows_a, rows_b, rows_c, acc_sh,
                 gsem_a, gsem_b, gsem_c, ssem_a, ssem_b, ssem_c):
    c = lax.axis_index("c")
    s = lax.axis_index("s")
    wid = s * _NCORES + c

    @pl.when(s < _NZ)
    def _():
        row0 = s * _RPZ
        pltpu.sync_copy(zsum_hbm.at[pl.ds(row0, _RPZ), :],
                        acc_sh.at[pl.ds(row0, _RPZ), :])

    plsc.subcore_barrier()

    bufs = ((rows_a, gsem_a, ssem_a), (rows_b, gsem_b, ssem_b),
            (rows_c, gsem_c, ssem_c))

    def _group(g, _):
        pltpu.sync_copy(src_hbm.at[wid, g], sidx)
        pltpu.sync_copy(dst_hbm.at[wid, g], didx)
        pltpu.async_copy(table_hbm.at[sidx.at[0]], rows_a, gsem_a)
        pltpu.async_copy(table_hbm.at[sidx.at[1]], rows_b, gsem_b)
        for j in range(_GC):
            buf, gsem, ssem = bufs[j % 3]
            if j + 2 < _GC:
                nbuf, ngsem, nssem = bufs[(j + 2) % 3]
                if j >= 1:
                    pltpu.make_async_copy(
                        nbuf, acc_sh.at[didx.at[j - 1]], nssem).wait()
                pltpu.async_copy(table_hbm.at[sidx.at[j + 2]], nbuf, ngsem)
            pltpu.make_async_copy(table_hbm.at[sidx.at[j]], buf, gsem).wait()
            pltpu.async_copy(buf, acc_sh.at[didx.at[j]], ssem, add=True)
        for j in range(_GC - 3, _GC):
            buf, _, ssem = bufs[j % 3]
            pltpu.make_async_copy(buf, acc_sh.at[didx.at[j]], ssem).wait()
        return 0

    lax.fori_loop(0, _NGROUP, _group, 0)
    plsc.subcore_barrier()

    @pl.when(s < _NZ)
    def _():
        row0 = s * _RPZ
        pltpu.sync_copy(acc_sh.at[pl.ds(row0, _RPZ), :],
                        sums_hbm.at[c, pl.ds(row0, _RPZ), :])


@jax.jit
def _segsum(table, src3, dst3, zsum):
    mesh = plsc.VectorSubcoreMesh(core_axis_name="c", subcore_axis_name="s")
    return pl.kernel(
        _segsum_body,
        out_type=jax.ShapeDtypeStruct((_NCORES, _N, _H), jnp.float32),
        mesh=mesh,
        scratch_types=[
            pltpu.VMEM((_GC, _K), jnp.int32),
            pltpu.VMEM((_GC, _K), jnp.int32),
            pltpu.VMEM((_K, _H), jnp.float32),
            pltpu.VMEM((_K, _H), jnp.float32),
            pltpu.VMEM((_K, _H), jnp.float32),
            pltpu.VMEM_SHARED((_N, _H), jnp.float32),
        ] + [pltpu.SemaphoreType.DMA] * 6,
    )(table, src3, dst3, zsum)


def _counts_body(dstv_hbm, dstc_hbm, zcnt_hbm,
                 cntv_hbm, cntc_hbm,
                 didx, ones, cnt_sh, stg_sem, csem):
    c = lax.axis_index("c")
    s = lax.axis_index("s")
    wid = s * _NCORES + c

    for i in range(_KC // 16):
        ones[pl.ds(i * 16, 16)] = jnp.ones((16,), jnp.float32)

    for d_hbm, out_hbm in ((dstv_hbm, cntv_hbm), (dstc_hbm, cntc_hbm)):
        @pl.when(s == 0)
        def _():
            pltpu.sync_copy(zcnt_hbm, cnt_sh)

        pltpu.async_copy(d_hbm.at[wid], didx, stg_sem)
        pltpu.make_async_copy(d_hbm.at[wid], didx, stg_sem).wait()
        plsc.subcore_barrier()

        def _fire(j, _):
            pltpu.async_copy(ones, cnt_sh.at[didx.at[j]], csem, add=True)
            return 0

        def _fire_drain(j, _):
            pltpu.async_copy(ones, cnt_sh.at[didx.at[j]], csem, add=True)
            pltpu.make_async_copy(ones, cnt_sh.at[didx.at[j - 8]], csem).wait()
            return 0

        lax.fori_loop(0, 8, _fire, 0)
        lax.fori_loop(8, _CPT, _fire_drain, 0)
        lax.fori_loop(_CPT - 8, _CPT, lambda j, _: (
            pltpu.make_async_copy(ones, cnt_sh.at[didx.at[j]], csem).wait(),
            0)[1], 0)
        plsc.subcore_barrier()

        @pl.when(s == 0)
        def _():
            pltpu.sync_copy(cnt_sh, out_hbm.at[c])

        plsc.subcore_barrier()


@jax.jit
def _counts(dstv4, dstc4, zcnt):
    mesh = plsc.VectorSubcoreMesh(core_axis_name="c", subcore_axis_name="s")
    return pl.kernel(
        _counts_body,
        out_type=(jax.ShapeDtypeStruct((_NCORES, _NP), jnp.float32),
                  jax.ShapeDtypeStruct((_NCORES, _NP), jnp.float32)),
        mesh=mesh,
        scratch_types=[
            pltpu.VMEM((_CPT, _KC), jnp.int32),
            pltpu.VMEM((_KC,), jnp.float32),
            pltpu.VMEM_SHARED((_NP,), jnp.float32),
            pltpu.SemaphoreType.DMA,
            pltpu.SemaphoreType.DMA,
        ],
    )(dstv4, dstc4, zcnt)


_BLK = 2000
_GRID = _N // _BLK


def _ln(x, g, b):
    m = jnp.mean(x, axis=-1, keepdims=True)
    v = jnp.mean((x - m) ** 2, axis=-1, keepdims=True)
    return (x - m) * jax.lax.rsqrt(v + 1e-5) * g + b


def _proj_body(xvar, xcon, wvT, bv, gv, bvn, wcT, bc, gc, bcn, xv_o, xc_o):
    hv = jax.nn.relu(jnp.dot(xvar[...], wvT[...],
                             preferred_element_type=jnp.float32) + bv[...])
    xv_o[...] = _ln(hv, gv[...], bvn[...])
    hc = jax.nn.relu(jnp.dot(xcon[...], wcT[...],
                             preferred_element_type=jnp.float32) + bc[...])
    xc_o[...] = _ln(hc, gc[...], bcn[...])


def _v2c_body(sums, cnts, xcon, xc, wmT, bm, wuT, wub, bu, gl, bl,
              wg1T, wg2T, bg, out):
    sp = sums[...]
    S = sp[0] + sp[1]
    cp = cnts[...]
    cnt = cp[0] + cp[1]
    agg = (jnp.dot(S, wmT[...], preferred_element_type=jnp.float32)
           + cnt * bm[...]) / (cnt + 1e-6)
    h = (jnp.dot(agg, wuT[...], preferred_element_type=jnp.float32)
         + xcon[...] * wub[...] + bu[...])
    con_new = _ln(jax.nn.relu(h), gl[...], bl[...])
    xc_v = xc[...]
    g = jax.nn.sigmoid(
        jnp.dot(xc_v, wg1T[...], preferred_element_type=jnp.float32)
        + jnp.dot(con_new, wg2T[...], preferred_element_type=jnp.float32)
        + bg[...])
    out[...] = g * con_new + (1.0 - g) * xc_v


def _c2v_body(sums, cnts, xv, wmT, bm, gl, bl, wg1T, wg2T, bg, out):
    sp = sums[...]
    S = sp[0] + sp[1]
    cp = cnts[...]
    cnt = cp[0] + cp[1]
    agg = (jnp.dot(S, wmT[...], preferred_element_type=jnp.float32)
           + cnt * bm[...]) / (cnt + 1e-6)
    var_new = _ln(jax.nn.relu(agg), gl[...], bl[...])
    xv_v = xv[...]
    g = jax.nn.sigmoid(
        jnp.dot(xv_v, wg1T[...], preferred_element_type=jnp.float32)
        + jnp.dot(var_new, wg2T[...], preferred_element_type=jnp.float32)
        + bg[...])
    out[...] = g * var_new + (1.0 - g) * xv_v


def _rows(blk_last):
    return pl.BlockSpec((_BLK, blk_last), lambda i: (i, 0))


def _prows(blk_last):
    return pl.BlockSpec((2, _BLK, blk_last), lambda i: (0, i, 0))


def _w(shape):
    return pl.BlockSpec(shape, lambda i: tuple(0 for _ in shape))


@jax.jit
def _tc_proj(x_var, x_con, wvT, bv, gv, bvn, wcT, bc, gc, bcn):
    return pl.pallas_call(
        _proj_body,
        grid=(_GRID,),
        in_specs=[_rows(4), _rows(1),
                  _w((4, _H)), _w((1, _H)), _w((1, _H)), _w((1, _H)),
                  _w((1, _H)), _w((1, _H)), _w((1, _H)), _w((1, _H))],
        out_specs=[_rows(_H), _rows(_H)],
        out_shape=[jax.ShapeDtypeStruct((_N, _H), jnp.float32),
                   jax.ShapeDtypeStruct((_N, _H), jnp.float32)],
    )(x_var, x_con, wvT, bv, gv, bvn, wcT, bc, gc, bcn)


@jax.jit
def _tc_v2c(sums, cnts, x_con, xc, wmT, bm, wuT, wub, bu, gl, bl,
            wg1T, wg2T, bg):
    return pl.pallas_call(
        _v2c_body,
        grid=(_GRID,),
        in_specs=[_prows(_H), _prows(1), _rows(1), _rows(_H),
                  _w((_H, _H)), _w((1, _H)), _w((_H, _H)), _w((1, _H)),
                  _w((1, _H)), _w((1, _H)), _w((1, _H)),
                  _w((_H, _H)), _w((_H, _H)), _w((1, _H))],
        out_specs=_rows(_H),
        out_shape=jax.ShapeDtypeStruct((_N, _H), jnp.float32),
    )(sums, cnts, x_con, xc, wmT, bm, wuT, wub, bu, gl, bl, wg1T, wg2T, bg)


@jax.jit
def _tc_c2v(sums, cnts, xv, wmT, bm, gl, bl, wg1T, wg2T, bg):
    return pl.pallas_call(
        _c2v_body,
        grid=(_GRID,),
        in_specs=[_prows(_H), _prows(1), _rows(_H),
                  _w((_H, _H)), _w((1, _H)), _w((1, _H)), _w((1, _H)),
                  _w((_H, _H)), _w((_H, _H)), _w((1, _H))],
        out_specs=_rows(_H),
        out_shape=jax.ShapeDtypeStruct((_N, _H), jnp.float32),
    )(sums, cnts, xv, wmT, bm, gl, bl, wg1T, wg2T, bg)


def kernel(x_var, x_con, e_var_con, e_con_var, params):
    p = params
    r2 = lambda b: b.reshape(1, _H)

    wvT = p["var_proj"][0].T
    bv = r2(p["var_proj"][1])
    gv, bvn = r2(p["var_norm"][0]), r2(p["var_norm"][1])
    wcT = p["con_proj"][0].T
    bc = r2(p["con_proj"][1])
    gc, bcn = r2(p["con_norm"][0]), r2(p["con_norm"][1])

    wmT = p["v2c_msg"][0].T
    bm = r2(p["v2c_msg"][1])
    wu = p["v2c_upd"][0]
    wuT = wu[:, :_H].T
    wub = wu[:, _H].reshape(1, _H)
    bu = r2(p["v2c_upd"][1])
    gl_c, bl_c = r2(p["v2c_ln"][0]), r2(p["v2c_ln"][1])
    wg_c = p["con_gate"][0]
    wg1T_c, wg2T_c = wg_c[:, :_H].T, wg_c[:, _H:].T
    bg_c = r2(p["con_gate"][1])

    wm2T = p["c2v_msg"][0].T
    bm2 = r2(p["c2v_msg"][1])
    gl_v, bl_v = r2(p["c2v_ln"][0]), r2(p["c2v_ln"][1])
    wg_v = p["var_gate"][0]
    wg1T_v, wg2T_v = wg_v[:, :_H].T, wg_v[:, _H:].T
    bg_v = r2(p["var_gate"][1])

    eshape = (_NW, _NGROUP, _GC, _K)
    src_v = e_var_con[0].astype(jnp.int32).reshape(eshape)
    dst_v = e_var_con[1].astype(jnp.int32).reshape(eshape)
    src_c = e_con_var[0].astype(jnp.int32).reshape(eshape)
    dst_c = e_con_var[1].astype(jnp.int32).reshape(eshape)
    zsum = jnp.zeros((_N, _H), jnp.float32)
    zcnt = jnp.zeros((_NP,), jnp.float32)

    pad = jnp.full((_EPAD,), _N, jnp.int32)
    cshape = (_NW, _CPT, _KC)
    dv4 = jnp.concatenate([e_var_con[1].astype(jnp.int32), pad]).reshape(cshape)
    dc4 = jnp.concatenate([e_con_var[1].astype(jnp.int32), pad]).reshape(cshape)
    cv, cc = _counts(dv4, dc4, zcnt)
    cnts_v = cv[:, :_N, None]
    cnts_c = cc[:, :_N, None]

    xv, xc = _tc_proj(x_var, x_con, wvT, bv, gv, bvn, wcT, bc, gc, bcn)

    for _ in range(2):
        sums = _segsum(xv, src_v, dst_v, zsum)
        xc = _tc_v2c(sums, cnts_v, x_con, xc,
                     wmT, bm, wuT, wub, bu, gl_c, bl_c,
                     wg1T_c, wg2T_c, bg_c)
        sums2 = _segsum(xc, src_c, dst_c, zsum)
        xv = _tc_c2v(sums2, cnts_c, xv,
                     wm2T, bm2, gl_v, bl_v, wg1T_v, wg2T_v, bg_v)
    return xv
